# baseline (speedup 1.0000x reference)
import jax
import jax.numpy as jnp
from jax.experimental import pallas as pl
from jax.experimental.pallas import tpu as pltpu


def _pos_kernel(row_ref, col_ref, out_ref, scratch, sems):
    b = out_ref.shape[0]
    f = col_ref.shape[1]
    h = row_ref.shape[0]
    w = col_ref.shape[0]
    n = h * w

    lane = jax.lax.broadcasted_iota(jnp.int32, (w, n), 1)
    sub = jax.lax.broadcasted_iota(jnp.int32, (w, n), 0)

    sel_x = (lane % w == sub).astype(jnp.float32)
    tcol = jnp.transpose(col_ref[...], (1, 0))
    scratch[0:f, :] = jax.lax.dot(
        tcol, sel_x, precision=jax.lax.Precision.HIGHEST
    )
    half = f // 2
    for i in range(b):
        for j in range(2):
            pltpu.make_async_copy(
                scratch.at[pl.ds(j * half, half)],
                out_ref.at[i, pl.ds(j * half, half)],
                sems.at[i, j],
            ).start()

    sel_y = (lane // w == sub).astype(jnp.float32)
    trow = jnp.transpose(row_ref[...], (1, 0))
    scratch[f : 2 * f, :] = jax.lax.dot(
        trow, sel_y, precision=jax.lax.Precision.HIGHEST
    )
    for i in range(b):
        for j in range(2):
            pltpu.make_async_copy(
                scratch.at[pl.ds(f + j * half, half)],
                out_ref.at[i, pl.ds(f + j * half, half)],
                sems.at[i, 2 + j],
            ).start()

    for i in range(b):
        for j in range(2):
            pltpu.make_async_copy(
                scratch.at[pl.ds(j * half, half)],
                out_ref.at[i, pl.ds(j * half, half)],
                sems.at[i, j],
            ).wait()
            pltpu.make_async_copy(
                scratch.at[pl.ds(f + j * half, half)],
                out_ref.at[i, pl.ds(f + j * half, half)],
                sems.at[i, 2 + j],
            ).wait()


def kernel(mask, row_embed, col_embed):
    b = mask.shape[0]
    h, w = mask.shape[-2], mask.shape[-1]
    f = col_embed.shape[-1]

    out = pl.pallas_call(
        _pos_kernel,
        in_specs=[
            pl.BlockSpec(memory_space=pltpu.MemorySpace.VMEM),
            pl.BlockSpec(memory_space=pltpu.MemorySpace.VMEM),
        ],
        out_specs=pl.BlockSpec(memory_space=pltpu.MemorySpace.HBM),
        out_shape=jax.ShapeDtypeStruct((b, 2 * f, h * w), jnp.float32),
        scratch_shapes=[
            pltpu.VMEM((2 * f, h * w), jnp.float32),
            pltpu.SemaphoreType.DMA((b, 4)),
        ],
    )(row_embed, col_embed)
    return out.reshape(b, 2 * f, h, w)
